# baseline (speedup 1.0000x reference)
import jax
import jax.numpy as jnp
from jax.experimental import pallas as pl
from jax.experimental.pallas import tpu as pltpu

_B = 4096
_P = 1024
_Q = 1024
_E = 8


def _moe_kernel(x_ref, w_ref, b_ref, rw_ref, out_ref, aux_ref, wgt_ref):
    e = pl.program_id(0)

    @pl.when(e == 0)
    def _router():
        logits = jnp.dot(x_ref[...], rw_ref[...],
                         preferred_element_type=jnp.float32)
        w = jax.nn.softmax(logits, axis=-1)
        wgt_ref[...] = w
        imp = jnp.mean(w, axis=0, keepdims=True)
        aux_ref[...] = jnp.float32(_E) * jnp.sum(imp * imp, keepdims=True)
        out_ref[...] = jnp.dot(w, b_ref[...],
                               preferred_element_type=jnp.float32)

    w_all = wgt_ref[...]
    mask = jax.lax.broadcasted_iota(jnp.int32, (1, _E), 1) == e
    wcol = jnp.sum(jnp.where(mask, w_all, 0.0), axis=1, keepdims=True)

    y = jnp.dot(x_ref[...], w_ref[0], preferred_element_type=jnp.float32)
    out_ref[...] = out_ref[...] + wcol * y


def kernel(inputs, expert_w, expert_b, router_w):
    x16 = inputs.astype(jnp.bfloat16)
    w16 = expert_w.astype(jnp.bfloat16)
    rw16 = router_w.astype(jnp.bfloat16)
    out, aux = pl.pallas_call(
        _moe_kernel,
        grid=(_E,),
        in_specs=[
            pl.BlockSpec((_B, _P), lambda e: (0, 0)),
            pl.BlockSpec((1, _P, _Q), lambda e: (e, 0, 0)),
            pl.BlockSpec((_E, _Q), lambda e: (0, 0)),
            pl.BlockSpec((_P, _E), lambda e: (0, 0)),
        ],
        out_specs=[
            pl.BlockSpec((_B, _Q), lambda e: (0, 0)),
            pl.BlockSpec((1, 1), lambda e: (0, 0)),
        ],
        out_shape=[
            jax.ShapeDtypeStruct((_B, _Q), jnp.float32),
            jax.ShapeDtypeStruct((1, 1), jnp.float32),
        ],
        scratch_shapes=[pltpu.VMEM((_B, _E), jnp.float32)],
        compiler_params=pltpu.CompilerParams(
            dimension_semantics=("arbitrary",),
        ),
    )(x16, w16, expert_b, rw16)
    return out, aux.reshape(())


# speedup vs baseline: 1.2591x; 1.2591x over previous
import jax
import jax.numpy as jnp
from jax.experimental import pallas as pl
from jax.experimental.pallas import tpu as pltpu

_B = 4096
_P = 1024
_Q = 1024
_E = 8


def _moe_kernel(x_ref, w_ref, b_ref, rw_ref, out_ref, aux_ref, wgt_ref):
    e = pl.program_id(0)

    @pl.when(e == 0)
    def _router():
        logits = jnp.dot(x_ref[...], rw_ref[...],
                         preferred_element_type=jnp.float32)
        w = jax.nn.softmax(logits, axis=-1)
        wgt_ref[...] = w
        imp = jnp.mean(w, axis=0, keepdims=True)
        aux_ref[...] = jnp.float32(_E) * jnp.sum(imp * imp, keepdims=True)
        out_ref[...] = jnp.dot(w, b_ref[...],
                               preferred_element_type=jnp.float32)

    w_all = wgt_ref[...]
    mask = jax.lax.broadcasted_iota(jnp.int32, (1, _E), 1) == e
    wcol = jnp.sum(jnp.where(mask, w_all, 0.0), axis=1, keepdims=True)

    _NC = _Q // 2
    for c in range(2):
        y = jnp.dot(x_ref[...], w_ref[0, :, c * _NC:(c + 1) * _NC],
                    preferred_element_type=jnp.float32)
        out_ref[:, c * _NC:(c + 1) * _NC] = (
            out_ref[:, c * _NC:(c + 1) * _NC] + wcol * y)


def kernel(inputs, expert_w, expert_b, router_w):
    out, aux = pl.pallas_call(
        _moe_kernel,
        grid=(_E,),
        in_specs=[
            pl.BlockSpec((_B, _P), lambda e: (0, 0)),
            pl.BlockSpec((1, _P, _Q), lambda e: (e, 0, 0)),
            pl.BlockSpec((_E, _Q), lambda e: (0, 0)),
            pl.BlockSpec((_P, _E), lambda e: (0, 0)),
        ],
        out_specs=[
            pl.BlockSpec((_B, _Q), lambda e: (0, 0)),
            pl.BlockSpec((1, 1), lambda e: (0, 0)),
        ],
        out_shape=[
            jax.ShapeDtypeStruct((_B, _Q), jnp.float32),
            jax.ShapeDtypeStruct((1, 1), jnp.float32),
        ],
        scratch_shapes=[pltpu.VMEM((_B, _E), jnp.float32)],
        compiler_params=pltpu.CompilerParams(
            dimension_semantics=("arbitrary",),
        ),
    )(inputs, expert_w, expert_b, router_w)
    return out, aux.reshape(())


# speedup vs baseline: 1.2628x; 1.0029x over previous
import jax
import jax.numpy as jnp
from jax.experimental import pallas as pl
from jax.experimental.pallas import tpu as pltpu

_B = 4096
_P = 1024
_Q = 1024
_E = 8


def _moe_kernel(x_ref, w_ref, b_ref, rw_ref, out_ref, aux_ref, wgt_ref):
    e = pl.program_id(0)

    @pl.when(e == 0)
    def _router():
        logits = jnp.dot(x_ref[...], rw_ref[...],
                         preferred_element_type=jnp.float32)
        w = jax.nn.softmax(logits, axis=-1)
        wgt_ref[...] = w
        imp = jnp.mean(w, axis=0, keepdims=True)
        aux_ref[...] = jnp.float32(_E) * jnp.sum(imp * imp, keepdims=True)
        out_ref[...] = jnp.dot(w, b_ref[...],
                               preferred_element_type=jnp.float32)

    w_all = wgt_ref[...]
    mask = jax.lax.broadcasted_iota(jnp.int32, (1, _E), 1) == e
    wcol = jnp.sum(jnp.where(mask, w_all, 0.0), axis=1, keepdims=True)

    y = jnp.dot(x_ref[...], w_ref[0], preferred_element_type=jnp.float32)
    out_ref[...] = out_ref[...] + wcol * y


def kernel(inputs, expert_w, expert_b, router_w):
    out, aux = pl.pallas_call(
        _moe_kernel,
        grid=(_E,),
        in_specs=[
            pl.BlockSpec((_B, _P), lambda e: (0, 0)),
            pl.BlockSpec((1, _P, _Q), lambda e: (e, 0, 0)),
            pl.BlockSpec((_E, _Q), lambda e: (0, 0)),
            pl.BlockSpec((_P, _E), lambda e: (0, 0)),
        ],
        out_specs=[
            pl.BlockSpec((_B, _Q), lambda e: (0, 0)),
            pl.BlockSpec((1, 1), lambda e: (0, 0)),
        ],
        out_shape=[
            jax.ShapeDtypeStruct((_B, _Q), jnp.float32),
            jax.ShapeDtypeStruct((1, 1), jnp.float32),
        ],
        scratch_shapes=[pltpu.VMEM((_B, _E), jnp.float32)],
        compiler_params=pltpu.CompilerParams(
            dimension_semantics=("arbitrary",),
        ),
    )(inputs, expert_w, expert_b, router_w)
    return out, aux.reshape(())
